# baseline (speedup 1.0000x reference)
import dataclasses
import functools

import numpy as np
import jax
import jax.numpy as jnp
from jax import lax
from jax.experimental import pallas as pl
from jax.experimental.pallas import tpu as pltpu
from jax.experimental.pallas import tpu_sc as plsc

GRID = 200
XMIN, XMAX = -5.0, 5.0
DX = (XMAX - XMIN) / GRID
LOG_DXDY = float(np.log(DX) + np.log(DX))
INV_STEP = float((GRID - 1) / (XMAX - XMIN))

NC, NS, L = 2, 16, 16
NW = NC * NS
B = 16384
BPW = B // NW
GCH = 128
NG = BPW // GCH


def _sc_gather_partials(tb_t, E):
    mesh = plsc.VectorSubcoreMesh(core_axis_name="c", subcore_axis_name="s")

    @functools.partial(
        pl.kernel,
        out_type=jax.ShapeDtypeStruct((NW, L), jnp.float32),
        mesh=mesh,
        scratch_types=[
            pltpu.VMEM((BPW,), jnp.float32),
            pltpu.VMEM((BPW,), jnp.float32),
            pltpu.VMEM((NG, GCH), jnp.int32),
            pltpu.VMEM((BPW,), jnp.float32),
            pltpu.VMEM((L,), jnp.float32),
            pltpu.SemaphoreType.DMA,
            pltpu.SemaphoreType.DMA,
            pltpu.SemaphoreType.DMA,
        ],
    )
    def k(tb_hbm, e_hbm, out_hbm, xv, yv, idxv, valv, accv,
          sem_a, sem_b, sem_g):
        wid = lax.axis_index("s") * NC + lax.axis_index("c")
        base = wid * BPW
        half = BPW // 2
        in_copies = []
        for h, sem in ((0, sem_a), (1, sem_b)):
            in_copies.append((
                pltpu.async_copy(tb_hbm.at[0, pl.ds(base + h * half, half)],
                                 xv.at[pl.ds(h * half, half)], sem),
                pltpu.async_copy(tb_hbm.at[1, pl.ds(base + h * half, half)],
                                 yv.at[pl.ds(h * half, half)], sem),
            ))
        accv[...] = jnp.zeros((L,), jnp.float32)
        hi = float(GRID - 1)
        gathers = []
        for c in range(NG):
            if c % (NG // 2) == 0:
                cx, cy = in_copies[c // (NG // 2)]
                cx.wait()
                cy.wait()
            for i in range(GCH // L):
                off = c * GCH + i * L
                fx = (xv[pl.ds(off, L)] - XMIN) * INV_STEP
                fy = (yv[pl.ds(off, L)] - XMIN) * INV_STEP
                fx = jnp.minimum(jnp.maximum(fx, 0.0), hi) + 0.5
                fy = jnp.minimum(jnp.maximum(fy, 0.0), hi) + 0.5
                ix = fx.astype(jnp.int32)
                iy = fy.astype(jnp.int32)
                idxv[c, pl.ds(i * L, L)] = ix * GRID + iy
            gathers.append(
                pltpu.async_copy(e_hbm.at[idxv.at[c]],
                                 valv.at[pl.ds(c * GCH, GCH)], sem_g))
        for c in range(NG):
            gathers[c].wait()
            for i in range(GCH // L):
                accv[...] = accv[...] + valv[pl.ds(c * GCH + i * L, L)]
        pltpu.sync_copy(accv, out_hbm.at[wid])

    return k(tb_t, E)


def _tc_combine(e2d, partials):
    def body(e_ref, p_ref, o_ref):
        se = jnp.sum(jnp.exp(-e_ref[...]))
        mean = jnp.sum(p_ref[...]) * (1.0 / B)
        val = -mean - jnp.log(se) - LOG_DXDY
        o_ref[...] = jnp.reshape(val, (1, 1))

    return pl.pallas_call(
        body,
        out_shape=jax.ShapeDtypeStruct((1, 1), jnp.float32),
    )(e2d, partials)


def kernel(train_batch, E):
    tb_t = train_batch.T
    partials = _sc_gather_partials(tb_t, E)
    out = _tc_combine(E.reshape(GRID, GRID), partials)
    return out[0, 0]


# speedup vs baseline: 1.0070x; 1.0070x over previous
import dataclasses
import functools

import numpy as np
import jax
import jax.numpy as jnp
from jax import lax
from jax.experimental import pallas as pl
from jax.experimental.pallas import tpu as pltpu
from jax.experimental.pallas import tpu_sc as plsc

GRID = 200
XMIN, XMAX = -5.0, 5.0
DX = (XMAX - XMIN) / GRID
LOG_DXDY = float(np.log(DX) + np.log(DX))
INV_STEP = float((GRID - 1) / (XMAX - XMIN))

NC, NS, L = 1, 16, 16
NW = NC * NS
B = 16384
BPW = B // NW
GCH = 128
NG = BPW // GCH


def _sc_gather_partials(tb_t, E):
    mesh = plsc.VectorSubcoreMesh(core_axis_name="c", subcore_axis_name="s",
                                  num_cores=NC)

    @functools.partial(
        pl.kernel,
        out_type=jax.ShapeDtypeStruct((NW, L), jnp.float32),
        mesh=mesh,
        scratch_types=[
            pltpu.VMEM((BPW,), jnp.float32),
            pltpu.VMEM((BPW,), jnp.float32),
            pltpu.VMEM((NG, GCH), jnp.int32),
            pltpu.VMEM((BPW,), jnp.float32),
            pltpu.VMEM((L,), jnp.float32),
            pltpu.SemaphoreType.DMA,
            pltpu.SemaphoreType.DMA,
            pltpu.SemaphoreType.DMA,
        ],
    )
    def k(tb_hbm, e_hbm, out_hbm, xv, yv, idxv, valv, accv,
          sem_a, sem_b, sem_g):
        wid = lax.axis_index("s") * NC + lax.axis_index("c")
        base = wid * BPW
        half = BPW // 2
        in_copies = []
        for h, sem in ((0, sem_a), (1, sem_b)):
            in_copies.append((
                pltpu.async_copy(tb_hbm.at[0, pl.ds(base + h * half, half)],
                                 xv.at[pl.ds(h * half, half)], sem),
                pltpu.async_copy(tb_hbm.at[1, pl.ds(base + h * half, half)],
                                 yv.at[pl.ds(h * half, half)], sem),
            ))
        accv[...] = jnp.zeros((L,), jnp.float32)
        hi = float(GRID - 1)
        gathers = []
        for c in range(NG):
            if c % (NG // 2) == 0:
                cx, cy = in_copies[c // (NG // 2)]
                cx.wait()
                cy.wait()
            for i in range(GCH // L):
                off = c * GCH + i * L
                fx = (xv[pl.ds(off, L)] - XMIN) * INV_STEP
                fy = (yv[pl.ds(off, L)] - XMIN) * INV_STEP
                fx = jnp.minimum(jnp.maximum(fx, 0.0), hi) + 0.5
                fy = jnp.minimum(jnp.maximum(fy, 0.0), hi) + 0.5
                ix = fx.astype(jnp.int32)
                iy = fy.astype(jnp.int32)
                idxv[c, pl.ds(i * L, L)] = ix * GRID + iy
            gathers.append(
                pltpu.async_copy(e_hbm.at[idxv.at[c]],
                                 valv.at[pl.ds(c * GCH, GCH)], sem_g))
        for c in range(NG):
            gathers[c].wait()
            for i in range(GCH // L):
                accv[...] = accv[...] + valv[pl.ds(c * GCH + i * L, L)]
        pltpu.sync_copy(accv, out_hbm.at[wid])

    return k(tb_t, E)


def _tc_combine(e2d, partials):
    def body(e_ref, p_ref, o_ref):
        se = jnp.sum(jnp.exp(-e_ref[...]))
        mean = jnp.sum(p_ref[...]) * (1.0 / B)
        val = -mean - jnp.log(se) - LOG_DXDY
        o_ref[...] = jnp.reshape(val, (1, 1))

    return pl.pallas_call(
        body,
        out_shape=jax.ShapeDtypeStruct((1, 1), jnp.float32),
    )(e2d, partials)


def kernel(train_batch, E):
    tb_t = train_batch.T
    partials = _sc_gather_partials(tb_t, E)
    out = _tc_combine(E.reshape(GRID, GRID), partials)
    return out[0, 0]
